# baseline (speedup 1.0000x reference)
import functools

import jax
import jax.numpy as jnp
from jax import lax
from jax.experimental import pallas as pl
from jax.experimental.pallas import tpu as pltpu
from jax.experimental.pallas import tpu_sc as plsc

_NUM_CORES = 2
_NUM_SUBCORES = 16
_NW = _NUM_CORES * _NUM_SUBCORES
_CHUNK = 16

_T = 77
_CLS0 = 5


def _sc_gather(table, idx):
    V, N, D = table.shape
    B = idx.shape[0]
    b_per_w = B // _NW
    n_chunks = b_per_w // _CHUNK
    mesh = plsc.VectorSubcoreMesh(core_axis_name="c", subcore_axis_name="s")

    @functools.partial(
        pl.kernel,
        mesh=mesh,
        out_type=jax.ShapeDtypeStruct((_T, B, D), jnp.float32),
        scratch_types=[
            pltpu.VMEM((b_per_w,), jnp.int32),
            pltpu.VMEM((_CHUNK, N, D), jnp.float32),
            pltpu.VMEM((_CHUNK, N, D), jnp.float32),
            pltpu.SemaphoreType.DMA,
            pltpu.SemaphoreType.DMA,
            pltpu.SemaphoreType.DMA,
            pltpu.SemaphoreType.DMA,
        ],
        compiler_params=pltpu.CompilerParams(use_tc_tiling_on_sc=True),
    )
    def k(table_hbm, idx_hbm, out_hbm, idx_v, rows_a, rows_b, g0, g1,
          w0, w1):
        wid = lax.axis_index("s") * _NUM_CORES + lax.axis_index("c")
        base = wid * b_per_w
        pltpu.sync_copy(idx_hbm.at[pl.ds(base, b_per_w)], idx_v)
        bufs, gsems, wsems = [rows_a, rows_b], [g0, g1], [w0, w1]
        pend_w = [[], []]

        def gather(ci, buf, sem):
            return pltpu.async_copy(
                table_hbm.at[idx_v.at[pl.ds(ci * _CHUNK, _CHUNK)]], buf,
                sem)

        pend_g = [gather(0, bufs[0], gsems[0]), None]
        for ci in range(n_chunks):
            cur, nxt = ci % 2, (ci + 1) % 2
            pend_g[cur].wait()
            if ci + 1 < n_chunks:
                for w in pend_w[nxt]:
                    w.wait()
                pend_w[nxt] = []
                pend_g[nxt] = gather(ci + 1, bufs[nxt], gsems[nxt])
            off = pl.multiple_of(base + ci * _CHUNK, _CHUNK)
            for j in range(N):
                pend_w[cur].append(pltpu.async_copy(
                    bufs[cur].at[:, j],
                    out_hbm.at[_CLS0 + j, pl.ds(off, _CHUNK)],
                    wsems[cur]))
        for lst in pend_w:
            for w in lst:
                w.wait()

    return k(table, idx)


_BB = 4096


def _tc_body(buf_ref, static_ref, vl_ref, tl_ref, other_ref, out_ref):
    del buf_ref
    bb = out_ref.shape[1]
    t = pl.program_id(0)
    p = t + jnp.where(t >= _CLS0, 4, 0)
    base = jnp.broadcast_to(static_ref[pl.ds(0, 1), pl.ds(p, 1), :],
                            (1, bb, 512))
    alt_v = jnp.broadcast_to(other_ref[pl.ds(0, 1), pl.ds(1, 1), :],
                             (1, bb, 512))
    alt_t = jnp.broadcast_to(other_ref[pl.ds(0, 1), pl.ds(3, 1), :],
                             (1, bb, 512))
    vmask = jnp.broadcast_to((vl_ref[...] != 0) & (p == 11), (1, bb, 512))
    tmask = jnp.broadcast_to((tl_ref[...] != 0) & (p == 14), (1, bb, 512))
    out_ref[...] = jnp.where(tmask, alt_t, jnp.where(vmask, alt_v, base))


def _tc_assemble(buf, static_rows, vl, tl, other):
    B = buf.shape[1]

    def out_map(t):
        return (t + jnp.where(t >= _CLS0, 4, 0), 0, 0)

    out_t = pl.pallas_call(
        _tc_body,
        grid=(_T - 4,),
        in_specs=[
            pl.BlockSpec(memory_space=pl.ANY),
            pl.BlockSpec((1, _T, 512), lambda t: (0, 0, 0)),
            pl.BlockSpec((1, _BB, 1), lambda t: (0, 0, 0)),
            pl.BlockSpec((1, _BB, 1), lambda t: (0, 0, 0)),
            pl.BlockSpec((1, 4, 512), lambda t: (0, 0, 0)),
        ],
        out_specs=pl.BlockSpec((1, _BB, 512), out_map),
        out_shape=jax.ShapeDtypeStruct((_T, B, 512), jnp.float32),
        input_output_aliases={0: 0},
    )(buf, static_rows, vl.reshape(1, B, 1), tl.reshape(1, B, 1), other)
    return jnp.transpose(out_t, (1, 0, 2))


def kernel(label, view_label, time_label, ctx_cls, token_prefix,
           token_suffix1, token_suffix2, token_suffix3, token_other):
    buf = _sc_gather(ctx_cls, label.astype(jnp.int32))
    static_rows = jnp.concatenate([
        token_prefix,
        jnp.zeros((1, 4, 512), jnp.float32),
        token_suffix1,
        token_other[:, 0:1],
        token_suffix2,
        token_other[:, 2:3],
        token_suffix3,
    ], axis=1)
    return _tc_assemble(buf, static_rows, view_label.astype(jnp.int32),
                        time_label.astype(jnp.int32), token_other)
